# baseline (speedup 1.0000x reference)
import functools

import jax
import jax.numpy as jnp
from jax import lax
from jax.experimental import pallas as pl
from jax.experimental.pallas import tpu as pltpu
from jax.experimental.pallas import tpu_sc as plsc

NVAR = 50000
NCSTR = 25000
NEDGE = 800000
NB = 16
H = 64
CFG = 32
NVP = 50176
NCP = 25088
EPAD = 819200
NC, NS, LANES = 2, 16, 16
NW = NC * NS
EB = 128
RB = 512
_NEG = -1e30


@functools.lru_cache(maxsize=None)
def _make_edge_pass(d):
    per_w = EPAD // NW
    nblk = per_w // EB
    cnk = 8
    nchk = nblk // cnk
    rps = NCP // NS
    zrows = 49
    mesh = plsc.VectorSubcoreMesh(
        core_axis_name="c", subcore_axis_name="s",
        num_cores=NC, num_subcores=NS)

    def body(table, gidx, sidx, ew, out, gi_buf, si_buf, ew_buf, rows3,
             acc, gsem0, gsem1, gsem2, ssem0, ssem1, ssem2):
        c = lax.axis_index("c")
        s = lax.axis_index("s")
        gsems = (gsem0, gsem1, gsem2)
        ssems = (ssem0, ssem1, ssem2)
        zvec = jnp.zeros((LANES,), jnp.float32)

        def zrow(r, _):
            for kk in range(d // LANES):
                rows3[0, r, pl.ds(kk * LANES, LANES)] = zvec
            return 0
        lax.fori_loop(0, zrows, zrow, 0)
        zsrc = rows3.at[0].at[pl.ds(0, zrows)]

        def zcp(i, _):
            pltpu.sync_copy(zsrc, acc.at[pl.ds(s * rps + i * zrows, zrows)])
            return 0
        lax.fori_loop(0, rps // zrows, zcp, 0)

        rbase = (c * NS + s) * nblk
        plsc.subcore_barrier()

        def start_gather(b, j):
            pltpu.async_copy(table.at[gi_buf.at[j]], rows3.at[b], gsems[b])

        def wait_gather(b, j):
            pltpu.make_async_copy(table.at[gi_buf.at[j]], rows3.at[b],
                                  gsems[b]).wait()

        def start_scatter(b, j):
            pltpu.async_copy(rows3.at[b], acc.at[si_buf.at[j]], ssems[b],
                             add=True)

        def wait_scatter(b, j):
            pltpu.make_async_copy(rows3.at[b], acc.at[si_buf.at[j]],
                                  ssems[b]).wait()

        def scale(b, j):
            def esc(q, _):
                jb = lax.broadcast(j, (LANES,))
                for u in range(4):
                    e = 4 * q + u
                    eb = lax.broadcast(e, (LANES,))
                    ewb = plsc.load_gather(ew_buf, [jb, eb])
                    for kk in range(d // LANES):
                        sl = pl.ds(kk * LANES, LANES)
                        rows3[b, e, sl] = rows3[b, e, sl] * ewb
                return 0
            lax.fori_loop(0, EB // 4, esc, 0)

        def chunk(ci, _):
            crow = rbase + ci * cnk
            pltpu.sync_copy(gidx.at[pl.ds(crow, cnk)], gi_buf)
            pltpu.sync_copy(sidx.at[pl.ds(crow, cnk)], si_buf)
            pltpu.sync_copy(ew.at[pl.ds(crow, cnk)], ew_buf)
            start_gather(0, 0)
            for j in range(cnk):
                b = j % 3
                nb_buf = (j + 1) % 3
                if j >= 2:
                    wait_scatter(nb_buf, j - 2)
                if j + 1 < cnk:
                    start_gather(nb_buf, j + 1)
                wait_gather(b, j)
                scale(b, j)
                start_scatter(b, j)
            wait_scatter((cnk - 2) % 3, cnk - 2)
            wait_scatter((cnk - 1) % 3, cnk - 1)
            return 0
        lax.fori_loop(0, nchk, chunk, 0)
        plsc.subcore_barrier()
        pltpu.sync_copy(acc.at[pl.ds(s * rps, rps)],
                        out.at[c].at[pl.ds(s * rps, rps)])

    return pl.kernel(
        body,
        out_type=jax.ShapeDtypeStruct((NC, NCP, d), jnp.float32),
        mesh=mesh,
        compiler_params=pltpu.CompilerParams(
            needs_layout_passes=False, use_tc_tiling_on_sc=False),
        scratch_types=[
            pltpu.VMEM((cnk, EB), jnp.int32),
            pltpu.VMEM((cnk, EB), jnp.int32),
            pltpu.VMEM((cnk, EB), jnp.float32),
            pltpu.VMEM((3, EB, d), jnp.float32),
            pltpu.VMEM_SHARED((NCP, d), jnp.float32),
            pltpu.SemaphoreType.DMA,
            pltpu.SemaphoreType.DMA,
            pltpu.SemaphoreType.DMA,
            pltpu.SemaphoreType.DMA,
            pltpu.SemaphoreType.DMA,
            pltpu.SemaphoreType.DMA,
        ],
    )


def _edge_pass_call(d, table, gidx, sidx, ew):
    return _make_edge_pass(d)(table, gidx, sidx, ew)


def _stats_call(x, d):
    nb = x.shape[0] // RB

    def body(x_ref, o_ref):
        i = pl.program_id(0)
        blk = x_ref[...]
        s = jnp.sum(blk, axis=0, keepdims=True)
        sq = jnp.sum(blk * blk, axis=0, keepdims=True)
        part = jnp.concatenate([s, sq, jnp.zeros((6, d), jnp.float32)], 0)

        @pl.when(i == 0)
        def _():
            o_ref[...] = part

        @pl.when(i != 0)
        def _():
            o_ref[...] = o_ref[...] + part

    return pl.pallas_call(
        body,
        grid=(nb,),
        in_specs=[pl.BlockSpec((RB, d), lambda i: (i, 0))],
        out_specs=pl.BlockSpec((8, d), lambda i: (0, 0)),
        out_shape=jax.ShapeDtypeStruct((8, d), jnp.float32),
    )(x)


def _norm_call(x, stats, g, b, wroot, n_true, d):
    npr = x.shape[0]
    nb = npr // RB
    inv_n = 1.0 / n_true

    def body(x_ref, st_ref, g_ref, b_ref, w_ref, xn_ref, rt_ref):
        i = pl.program_id(0)
        m = st_ref[0:1, :] * inv_n
        v = st_ref[1:2, :] * inv_n - m * m
        rs = lax.rsqrt(v + 1e-5)
        xn = (x_ref[...] - m) * rs * g_ref[...] + b_ref[...]
        rid = i * RB + lax.broadcasted_iota(jnp.int32, (RB, 1), 0)
        xn = jnp.where(rid < n_true, xn, 0.0)
        xn_ref[...] = xn
        rt_ref[...] = jnp.dot(xn, w_ref[...],
                              preferred_element_type=jnp.float32)

    return pl.pallas_call(
        body,
        grid=(nb,),
        in_specs=[
            pl.BlockSpec((RB, d), lambda i: (i, 0)),
            pl.BlockSpec((8, d), lambda i: (0, 0)),
            pl.BlockSpec((1, d), lambda i: (0, 0)),
            pl.BlockSpec((1, d), lambda i: (0, 0)),
            pl.BlockSpec((d, H), lambda i: (0, 0)),
        ],
        out_specs=[
            pl.BlockSpec((RB, d), lambda i: (i, 0)),
            pl.BlockSpec((RB, H), lambda i: (i, 0)),
        ],
        out_shape=[
            jax.ShapeDtypeStruct((npr, d), jnp.float32),
            jax.ShapeDtypeStruct((npr, H), jnp.float32),
        ],
    )(x, stats, g, b, wroot)


def _post_call(acc2, deg2, root, wrel, brel, n_true, d):
    npr = root.shape[0]
    nb = npr // RB
    nacc = NCP // RB

    def body(a0, a1, d0, d1, rt_ref, w_ref, bb_ref, y_ref):
        i = pl.program_id(0)
        valid = (i < nacc).astype(jnp.float32)
        agg = (a0[0] + a1[0]) * valid
        dg = (d0[0][:, 0:1] + d1[0][:, 0:1]) * valid
        agg = agg / jnp.maximum(dg, 1.0)
        y = (jnp.dot(agg, w_ref[...], preferred_element_type=jnp.float32)
             + bb_ref[...] + rt_ref[...])
        rid = i * RB + lax.broadcasted_iota(jnp.int32, (RB, 1), 0)
        y_ref[...] = jnp.where(rid < n_true, jnp.maximum(y, 0.0), 0.0)

    amap = lambda p: (lambda i: (p, jnp.minimum(i, nacc - 1), 0))
    return pl.pallas_call(
        body,
        grid=(nb,),
        in_specs=[
            pl.BlockSpec((1, RB, d), amap(0)),
            pl.BlockSpec((1, RB, d), amap(1)),
            pl.BlockSpec((1, RB, 16), amap(0)),
            pl.BlockSpec((1, RB, 16), amap(1)),
            pl.BlockSpec((RB, H), lambda i: (i, 0)),
            pl.BlockSpec((d, H), lambda i: (0, 0)),
            pl.BlockSpec((1, H), lambda i: (0, 0)),
        ],
        out_specs=pl.BlockSpec((RB, H), lambda i: (i, 0)),
        out_shape=jax.ShapeDtypeStruct((npr, H), jnp.float32),
    )(acc2, acc2, deg2, deg2, root, wrel, brel)


def _attg(xb, w1, b1, w2row, b2):
    h = jnp.maximum(jnp.dot(xb, w1, preferred_element_type=jnp.float32)
                    + b1, 0.0)
    return jnp.sum(h * w2row, axis=1, keepdims=True) + b2


def _poolA_call(x, batch, w1, b1, w2row, b2):
    npr = x.shape[0]
    nb = npr // RB

    def body(x_ref, bt_ref, w1_ref, b1_ref, w2_ref, b2_ref, xm_ref, gm_ref):
        i = pl.program_id(0)
        xb = x_ref[...]
        g = _attg(xb, w1_ref[...], b1_ref[...], w2_ref[...], b2_ref[...])
        gb = jnp.broadcast_to(g, (RB, H))
        bt = bt_ref[...]
        xms, gms = [], []
        for bb in range(NB):
            msk = bt == bb
            xms.append(jnp.max(jnp.where(msk, xb, _NEG), 0, keepdims=True))
            gms.append(jnp.max(jnp.where(msk, gb, _NEG), 0, keepdims=True))
        xm = jnp.concatenate(xms, 0)
        gm = jnp.concatenate(gms, 0)

        @pl.when(i == 0)
        def _():
            xm_ref[...] = xm
            gm_ref[...] = gm

        @pl.when(i != 0)
        def _():
            xm_ref[...] = jnp.maximum(xm_ref[...], xm)
            gm_ref[...] = jnp.maximum(gm_ref[...], gm)

    return pl.pallas_call(
        body,
        grid=(nb,),
        in_specs=[
            pl.BlockSpec((RB, H), lambda i: (i, 0)),
            pl.BlockSpec((RB, 1), lambda i: (i, 0)),
            pl.BlockSpec((H, H // 2), lambda i: (0, 0)),
            pl.BlockSpec((1, H // 2), lambda i: (0, 0)),
            pl.BlockSpec((1, H // 2), lambda i: (0, 0)),
            pl.BlockSpec((1, 1), lambda i: (0, 0)),
        ],
        out_specs=[
            pl.BlockSpec((NB, H), lambda i: (0, 0)),
            pl.BlockSpec((NB, H), lambda i: (0, 0)),
        ],
        out_shape=[
            jax.ShapeDtypeStruct((NB, H), jnp.float32),
            jax.ShapeDtypeStruct((NB, H), jnp.float32),
        ],
    )(x, batch, w1, b1, w2row, b2)


def _poolB_call(x, batch, w1, b1, w2row, b2, gmax):
    npr = x.shape[0]
    nb = npr // RB

    def body(x_ref, bt_ref, w1_ref, b1_ref, w2_ref, b2_ref, gm_ref,
             se_ref, sx_ref):
        i = pl.program_id(0)
        xb = x_ref[...]
        g = _attg(xb, w1_ref[...], b1_ref[...], w2_ref[...], b2_ref[...])
        bt = bt_ref[...]
        ses, sxs = [], []
        for bb in range(NB):
            msk = bt == bb
            e = jnp.where(msk, jnp.exp(g - gm_ref[bb, 0]), 0.0)
            ses.append(jnp.sum(e, 0, keepdims=True))
            sxs.append(jnp.sum(e * xb, 0, keepdims=True))
        se = jnp.broadcast_to(jnp.concatenate(ses, 0), (NB, H))
        sx = jnp.concatenate(sxs, 0)

        @pl.when(i == 0)
        def _():
            se_ref[...] = se
            sx_ref[...] = sx

        @pl.when(i != 0)
        def _():
            se_ref[...] = se_ref[...] + se
            sx_ref[...] = sx_ref[...] + sx

    return pl.pallas_call(
        body,
        grid=(nb,),
        in_specs=[
            pl.BlockSpec((RB, H), lambda i: (i, 0)),
            pl.BlockSpec((RB, 1), lambda i: (i, 0)),
            pl.BlockSpec((H, H // 2), lambda i: (0, 0)),
            pl.BlockSpec((1, H // 2), lambda i: (0, 0)),
            pl.BlockSpec((1, H // 2), lambda i: (0, 0)),
            pl.BlockSpec((1, 1), lambda i: (0, 0)),
            pl.BlockSpec((NB, H), lambda i: (0, 0)),
        ],
        out_specs=[
            pl.BlockSpec((NB, H), lambda i: (0, 0)),
            pl.BlockSpec((NB, H), lambda i: (0, 0)),
        ],
        out_shape=[
            jax.ShapeDtypeStruct((NB, H), jnp.float32),
            jax.ShapeDtypeStruct((NB, H), jnp.float32),
        ],
    )(x, batch, w1, b1, w2row, b2, gmax)


def _head_call(vmax, cmax, vse, vsx, cse, csx, hp):
    def body(vm, cm, vse_r, vsx_r, cse_r, csx_r, w1, b1, w2, b2,
             wmu, bmu, wsg, bsg, o_ref):
        vatt = vsx_r[...] / (vse_r[...] + 1e-16)
        catt = csx_r[...] / (cse_r[...] + 1e-16)
        emb = jnp.concatenate([vm[...], cm[...], vatt, catt], axis=1)
        h = jnp.maximum(jnp.dot(emb, w1[...],
                                preferred_element_type=jnp.float32)
                        + b1[...], 0.0)
        h = jnp.maximum(jnp.dot(h, w2[...],
                                preferred_element_type=jnp.float32)
                        + b2[...], 0.0)
        mu = jnp.dot(h, wmu[...], preferred_element_type=jnp.float32) + bmu[...]
        sg = jnp.exp(jnp.dot(h, wsg[...], preferred_element_type=jnp.float32)
                     + bsg[...])
        o_ref[0] = mu
        o_ref[1] = sg

    out = pl.pallas_call(
        body,
        out_shape=jax.ShapeDtypeStruct((2, NB, CFG), jnp.float32),
    )(vmax, cmax, vse, vsx, cse, csx,
      hp['W1'], hp['b1'][None], hp['W2'], hp['b2'][None],
      hp['Wmu'], hp['bmu'][None], hp['Wsig'], hp['bsig'][None])
    return jnp.stack([out[0], out[1]], axis=-1)


def _pad_rows_cols(a, rows, cols):
    return jnp.pad(a, ((0, rows - a.shape[0]), (0, cols - a.shape[1])))


def kernel(var_feats, cstr_feats, edge_attr, edge_index, var_batch_el,
           cstr_batch_el, params):
    xv = _pad_rows_cols(var_feats, NVP, 16)
    xc = _pad_rows_cols(cstr_feats, NCP, 16)
    srcp = jnp.pad(edge_index[0], (0, EPAD - NEDGE)).reshape(EPAD // EB, EB)
    dstp = jnp.pad(edge_index[1], (0, EPAD - NEDGE)).reshape(EPAD // EB, EB)
    ewp = jnp.pad(edge_attr, (0, EPAD - NEDGE)).reshape(EPAD // EB, EB)
    onesw = jnp.pad(jnp.ones((NEDGE,), jnp.float32),
                    (0, EPAD - NEDGE)).reshape(EPAD // EB, EB)
    ones_table = jnp.ones((NCP, 16), jnp.float32)

    degv2 = _edge_pass_call(16, ones_table, srcp, dstp, onesw)
    degc2 = _edge_pass_call(16, ones_table, dstp, srcp, onesw)

    for li, L in enumerate(params['layers']):
        d_v, d_c = xv.shape[1], xc.shape[1]
        if li == 0:
            ng = jnp.pad(L['nbn_g'], (0, 16 - 9), constant_values=1.0)[None]
            nb_ = jnp.pad(L['nbn_b'], (0, 16 - 9))[None]
            cg = jnp.pad(L['cbn_g'], (0, 16 - 1), constant_values=1.0)[None]
            cb_ = jnp.pad(L['cbn_b'], (0, 16 - 1))[None]
            nWroot = jnp.pad(L['nWroot'], ((0, 16 - 9), (0, 0)))
            cWroot = jnp.pad(L['cWroot'], ((0, 16 - 1), (0, 0)))
            nWrel = jnp.pad(L['nWrel'], ((0, 16 - 1), (0, 0)))
            cWrel = jnp.pad(L['cWrel'], ((0, 16 - 9), (0, 0)))
        else:
            ng, nb_ = L['nbn_g'][None], L['nbn_b'][None]
            cg, cb_ = L['cbn_g'][None], L['cbn_b'][None]
            nWroot, cWroot = L['nWroot'], L['cWroot']
            nWrel, cWrel = L['nWrel'], L['cWrel']

        stv = _stats_call(xv, d_v)
        stc = _stats_call(xc, d_c)
        xv_n, rootv = _norm_call(xv, stv, ng, nb_, nWroot, NVAR, d_v)
        xc_n, rootc = _norm_call(xc, stc, cg, cb_, cWroot, NCSTR, d_c)
        accv2 = _edge_pass_call(d_c, xc_n, srcp, dstp, ewp)
        accc2 = _edge_pass_call(d_v, xv_n, dstp, srcp, ewp)
        xv = _post_call(accv2, degv2, rootv, nWrel, L['nbrel'][None],
                        NVAR, d_c)
        xc = _post_call(accc2, degc2, rootc, cWrel, L['cbrel'][None],
                        NCSTR, d_v)

    vb = jnp.pad(var_batch_el, (0, NVP - NVAR), constant_values=NB)[:, None]
    cbt = jnp.pad(cstr_batch_el, (0, NCP - NCSTR), constant_values=NB)[:, None]
    av, ac = params['att_var'], params['att_cstr']
    aw2v = av['W2'][:, 0][None]
    aw2c = ac['W2'][:, 0][None]
    ab2v = av['b2'][None]
    ab2c = ac['b2'][None]
    vmax, vgm = _poolA_call(xv, vb, av['W1'], av['b1'][None], aw2v, ab2v)
    cmax, cgm = _poolA_call(xc, cbt, ac['W1'], ac['b1'][None], aw2c, ab2c)
    vse, vsx = _poolB_call(xv, vb, av['W1'], av['b1'][None], aw2v, ab2v, vgm)
    cse, csx = _poolB_call(xc, cbt, ac['W1'], ac['b1'][None], aw2c, ab2c, cgm)
    return _head_call(vmax, cmax, vse, vsx, cse, csx, params['head'])


# speedup vs baseline: 1.1420x; 1.1420x over previous
import functools

import jax
import jax.numpy as jnp
from jax import lax
from jax.experimental import pallas as pl
from jax.experimental.pallas import tpu as pltpu
from jax.experimental.pallas import tpu_sc as plsc

NVAR = 50000
NCSTR = 25000
NEDGE = 800000
NB = 16
H = 64
CFG = 32
NVP = 50176
NCP = 25088
EPAD = 819200
NC, NS, LANES = 2, 16, 16
NW = NC * NS
EB = 128
RB = 512
_NEG = -1e30


@functools.lru_cache(maxsize=None)
def _make_edge_pass(d):
    per_w = EPAD // NW
    nblk = per_w // EB
    cnk = 10 if d == 64 else 25
    nchk = nblk // cnk
    rps = NCP // NS
    zrows = 49
    mesh = plsc.VectorSubcoreMesh(
        core_axis_name="c", subcore_axis_name="s",
        num_cores=NC, num_subcores=NS)

    def body(table, gidx, sidx, ew, out, gi_buf, si_buf, ew_buf, rows3,
             acc, gsem0, gsem1, gsem2, ssem0, ssem1, ssem2):
        c = lax.axis_index("c")
        s = lax.axis_index("s")
        gsems = (gsem0, gsem1, gsem2)
        ssems = (ssem0, ssem1, ssem2)
        zvec = jnp.zeros((LANES,), jnp.float32)

        def zrow(r, _):
            for kk in range(d // LANES):
                rows3[0, r, pl.ds(kk * LANES, LANES)] = zvec
            return 0
        lax.fori_loop(0, zrows, zrow, 0)
        zsrc = rows3.at[0].at[pl.ds(0, zrows)]

        def zcp(i, _):
            pltpu.sync_copy(zsrc, acc.at[pl.ds(s * rps + i * zrows, zrows)])
            return 0
        lax.fori_loop(0, rps // zrows, zcp, 0)

        rbase = (c * NS + s) * nblk
        plsc.subcore_barrier()

        def start_gather(b, j):
            pltpu.async_copy(table.at[gi_buf.at[j]], rows3.at[b], gsems[b])

        def wait_gather(b, j):
            pltpu.make_async_copy(table.at[gi_buf.at[j]], rows3.at[b],
                                  gsems[b]).wait()

        def start_scatter(b, j):
            pltpu.async_copy(rows3.at[b], acc.at[si_buf.at[j]], ssems[b],
                             add=True)

        def wait_scatter(b, j):
            pltpu.make_async_copy(rows3.at[b], acc.at[si_buf.at[j]],
                                  ssems[b]).wait()

        def scale(b, j):
            def esc(q, _):
                jb = lax.broadcast(j, (LANES,))
                for u in range(4):
                    e = 4 * q + u
                    eb = lax.broadcast(e, (LANES,))
                    ewb = plsc.load_gather(ew_buf, [jb, eb])
                    for kk in range(d // LANES):
                        sl = pl.ds(kk * LANES, LANES)
                        rows3[b, e, sl] = rows3[b, e, sl] * ewb
                return 0
            lax.fori_loop(0, EB // 4, esc, 0)

        def chunk(ci, _):
            crow = rbase + ci * cnk
            pltpu.sync_copy(gidx.at[pl.ds(crow, cnk)], gi_buf)
            pltpu.sync_copy(sidx.at[pl.ds(crow, cnk)], si_buf)
            pltpu.sync_copy(ew.at[pl.ds(crow, cnk)], ew_buf)
            start_gather(0, 0)
            for j in range(cnk):
                b = j % 3
                nb_buf = (j + 1) % 3
                if j >= 2:
                    wait_scatter(nb_buf, j - 2)
                if j + 1 < cnk:
                    start_gather(nb_buf, j + 1)
                wait_gather(b, j)
                scale(b, j)
                start_scatter(b, j)
            wait_scatter((cnk - 2) % 3, cnk - 2)
            wait_scatter((cnk - 1) % 3, cnk - 1)
            return 0
        lax.fori_loop(0, nchk, chunk, 0)
        plsc.subcore_barrier()
        pltpu.sync_copy(acc.at[pl.ds(s * rps, rps)],
                        out.at[c].at[pl.ds(s * rps, rps)])

    return pl.kernel(
        body,
        out_type=jax.ShapeDtypeStruct((NC, NCP, d), jnp.float32),
        mesh=mesh,
        compiler_params=pltpu.CompilerParams(
            needs_layout_passes=False, use_tc_tiling_on_sc=False),
        scratch_types=[
            pltpu.VMEM((cnk, EB), jnp.int32),
            pltpu.VMEM((cnk, EB), jnp.int32),
            pltpu.VMEM((cnk, EB), jnp.float32),
            pltpu.VMEM((3, EB, d), jnp.float32),
            pltpu.VMEM_SHARED((NCP, d), jnp.float32),
            pltpu.SemaphoreType.DMA,
            pltpu.SemaphoreType.DMA,
            pltpu.SemaphoreType.DMA,
            pltpu.SemaphoreType.DMA,
            pltpu.SemaphoreType.DMA,
            pltpu.SemaphoreType.DMA,
        ],
    )


def _edge_pass_call(d, table, gidx, sidx, ew):
    return _make_edge_pass(d)(table, gidx, sidx, ew)


@functools.lru_cache(maxsize=None)
def _make_deg_pass():
    d = 16
    per_w = EPAD // NW
    nblk = per_w // EB
    cnk = 25
    nchk = nblk // cnk
    rps = NCP // NS
    zrows = 49
    mesh = plsc.VectorSubcoreMesh(
        core_axis_name="c", subcore_axis_name="s",
        num_cores=NC, num_subcores=NS)

    def body(sidx_a, sidx_b, out, ia_buf, ib_buf, ones_v, acc2, sem_a, sem_b):
        c = lax.axis_index("c")
        s = lax.axis_index("s")
        onev = jnp.ones((LANES,), jnp.float32)
        zvec = jnp.zeros((LANES,), jnp.float32)

        def orow(r, _):
            ones_v[r, pl.ds(0, LANES)] = onev
            return 0
        lax.fori_loop(0, EB, orow, 0)

        def zrow(r, _):
            ones_v[EB + r, pl.ds(0, LANES)] = zvec
            return 0
        lax.fori_loop(0, zrows, zrow, 0)
        zsrc = ones_v.at[pl.ds(EB, zrows)]

        def zcp(i, _):
            pltpu.sync_copy(zsrc,
                            acc2.at[0].at[pl.ds(s * rps + i * zrows, zrows)])
            pltpu.sync_copy(zsrc,
                            acc2.at[1].at[pl.ds(s * rps + i * zrows, zrows)])
            return 0
        lax.fori_loop(0, rps // zrows, zcp, 0)

        rbase = (c * NS + s) * nblk
        plsc.subcore_barrier()
        ones_rows = ones_v.at[pl.ds(0, EB)]

        def chunk(ci, _):
            crow = rbase + ci * cnk
            pltpu.sync_copy(sidx_a.at[pl.ds(crow, cnk)], ia_buf)
            pltpu.sync_copy(sidx_b.at[pl.ds(crow, cnk)], ib_buf)
            for j in range(cnk):
                if j >= 1:
                    pltpu.make_async_copy(
                        ones_rows, acc2.at[0].at[ia_buf.at[j - 1]],
                        sem_a).wait()
                    pltpu.make_async_copy(
                        ones_rows, acc2.at[1].at[ib_buf.at[j - 1]],
                        sem_b).wait()
                pltpu.async_copy(ones_rows, acc2.at[0].at[ia_buf.at[j]],
                                 sem_a, add=True)
                pltpu.async_copy(ones_rows, acc2.at[1].at[ib_buf.at[j]],
                                 sem_b, add=True)
            pltpu.make_async_copy(ones_rows, acc2.at[0].at[ia_buf.at[cnk - 1]],
                                  sem_a).wait()
            pltpu.make_async_copy(ones_rows, acc2.at[1].at[ib_buf.at[cnk - 1]],
                                  sem_b).wait()
            return 0
        lax.fori_loop(0, nchk, chunk, 0)
        plsc.subcore_barrier()
        pltpu.sync_copy(acc2.at[0].at[pl.ds(s * rps, rps)],
                        out.at[c].at[0].at[pl.ds(s * rps, rps)])
        pltpu.sync_copy(acc2.at[1].at[pl.ds(s * rps, rps)],
                        out.at[c].at[1].at[pl.ds(s * rps, rps)])

    return pl.kernel(
        body,
        out_type=jax.ShapeDtypeStruct((NC, 2, NCP, d), jnp.float32),
        mesh=mesh,
        compiler_params=pltpu.CompilerParams(
            needs_layout_passes=False, use_tc_tiling_on_sc=False),
        scratch_types=[
            pltpu.VMEM((cnk, EB), jnp.int32),
            pltpu.VMEM((cnk, EB), jnp.int32),
            pltpu.VMEM((EB + zrows, d), jnp.float32),
            pltpu.VMEM_SHARED((2, NCP, d), jnp.float32),
            pltpu.SemaphoreType.DMA,
            pltpu.SemaphoreType.DMA,
        ],
    )


def _deg_pass_call(dstp, srcp):
    return _make_deg_pass()(dstp, srcp)


def _stats_call(x, d):
    nb = x.shape[0] // RB

    def body(x_ref, o_ref):
        i = pl.program_id(0)
        blk = x_ref[...]
        s = jnp.sum(blk, axis=0, keepdims=True)
        sq = jnp.sum(blk * blk, axis=0, keepdims=True)
        part = jnp.concatenate([s, sq, jnp.zeros((6, d), jnp.float32)], 0)

        @pl.when(i == 0)
        def _():
            o_ref[...] = part

        @pl.when(i != 0)
        def _():
            o_ref[...] = o_ref[...] + part

    return pl.pallas_call(
        body,
        grid=(nb,),
        in_specs=[pl.BlockSpec((RB, d), lambda i: (i, 0))],
        out_specs=pl.BlockSpec((8, d), lambda i: (0, 0)),
        out_shape=jax.ShapeDtypeStruct((8, d), jnp.float32),
    )(x)


def _norm_call(x, stats, g, b, wroot, n_true, d):
    npr = x.shape[0]
    nb = npr // RB
    inv_n = 1.0 / n_true

    def body(x_ref, st_ref, g_ref, b_ref, w_ref, xn_ref, rt_ref):
        i = pl.program_id(0)
        m = st_ref[0:1, :] * inv_n
        v = st_ref[1:2, :] * inv_n - m * m
        rs = lax.rsqrt(v + 1e-5)
        xn = (x_ref[...] - m) * rs * g_ref[...] + b_ref[...]
        rid = i * RB + lax.broadcasted_iota(jnp.int32, (RB, 1), 0)
        xn = jnp.where(rid < n_true, xn, 0.0)
        xn_ref[...] = xn
        rt_ref[...] = jnp.dot(xn, w_ref[...],
                              preferred_element_type=jnp.float32)

    return pl.pallas_call(
        body,
        grid=(nb,),
        in_specs=[
            pl.BlockSpec((RB, d), lambda i: (i, 0)),
            pl.BlockSpec((8, d), lambda i: (0, 0)),
            pl.BlockSpec((1, d), lambda i: (0, 0)),
            pl.BlockSpec((1, d), lambda i: (0, 0)),
            pl.BlockSpec((d, H), lambda i: (0, 0)),
        ],
        out_specs=[
            pl.BlockSpec((RB, d), lambda i: (i, 0)),
            pl.BlockSpec((RB, H), lambda i: (i, 0)),
        ],
        out_shape=[
            jax.ShapeDtypeStruct((npr, d), jnp.float32),
            jax.ShapeDtypeStruct((npr, H), jnp.float32),
        ],
    )(x, stats, g, b, wroot)


def _post_call(acc2, deg2, root, wrel, brel, n_true, d):
    npr = root.shape[0]
    nb = npr // RB
    nacc = NCP // RB

    def body(a0, a1, d0, d1, rt_ref, w_ref, bb_ref, y_ref):
        i = pl.program_id(0)
        valid = (i < nacc).astype(jnp.float32)
        agg = (a0[0] + a1[0]) * valid
        dg = (d0[0][:, 0:1] + d1[0][:, 0:1]) * valid
        agg = agg / jnp.maximum(dg, 1.0)
        y = (jnp.dot(agg, w_ref[...], preferred_element_type=jnp.float32)
             + bb_ref[...] + rt_ref[...])
        rid = i * RB + lax.broadcasted_iota(jnp.int32, (RB, 1), 0)
        y_ref[...] = jnp.where(rid < n_true, jnp.maximum(y, 0.0), 0.0)

    amap = lambda p: (lambda i: (p, jnp.minimum(i, nacc - 1), 0))
    return pl.pallas_call(
        body,
        grid=(nb,),
        in_specs=[
            pl.BlockSpec((1, RB, d), amap(0)),
            pl.BlockSpec((1, RB, d), amap(1)),
            pl.BlockSpec((1, RB, 16), amap(0)),
            pl.BlockSpec((1, RB, 16), amap(1)),
            pl.BlockSpec((RB, H), lambda i: (i, 0)),
            pl.BlockSpec((d, H), lambda i: (0, 0)),
            pl.BlockSpec((1, H), lambda i: (0, 0)),
        ],
        out_specs=pl.BlockSpec((RB, H), lambda i: (i, 0)),
        out_shape=jax.ShapeDtypeStruct((npr, H), jnp.float32),
    )(acc2, acc2, deg2, deg2, root, wrel, brel)


def _attg(xb, w1, b1, w2row, b2):
    h = jnp.maximum(jnp.dot(xb, w1, preferred_element_type=jnp.float32)
                    + b1, 0.0)
    return jnp.sum(h * w2row, axis=1, keepdims=True) + b2


def _poolA_call(x, batch, w1, b1, w2row, b2):
    npr = x.shape[0]
    nb = npr // RB

    def body(x_ref, bt_ref, w1_ref, b1_ref, w2_ref, b2_ref, xm_ref, gm_ref):
        i = pl.program_id(0)
        xb = x_ref[...]
        g = _attg(xb, w1_ref[...], b1_ref[...], w2_ref[...], b2_ref[...])
        gb = jnp.broadcast_to(g, (RB, H))
        bt = bt_ref[...]
        xms, gms = [], []
        for bb in range(NB):
            msk = bt == bb
            xms.append(jnp.max(jnp.where(msk, xb, _NEG), 0, keepdims=True))
            gms.append(jnp.max(jnp.where(msk, gb, _NEG), 0, keepdims=True))
        xm = jnp.concatenate(xms, 0)
        gm = jnp.concatenate(gms, 0)

        @pl.when(i == 0)
        def _():
            xm_ref[...] = xm
            gm_ref[...] = gm

        @pl.when(i != 0)
        def _():
            xm_ref[...] = jnp.maximum(xm_ref[...], xm)
            gm_ref[...] = jnp.maximum(gm_ref[...], gm)

    return pl.pallas_call(
        body,
        grid=(nb,),
        in_specs=[
            pl.BlockSpec((RB, H), lambda i: (i, 0)),
            pl.BlockSpec((RB, 1), lambda i: (i, 0)),
            pl.BlockSpec((H, H // 2), lambda i: (0, 0)),
            pl.BlockSpec((1, H // 2), lambda i: (0, 0)),
            pl.BlockSpec((1, H // 2), lambda i: (0, 0)),
            pl.BlockSpec((1, 1), lambda i: (0, 0)),
        ],
        out_specs=[
            pl.BlockSpec((NB, H), lambda i: (0, 0)),
            pl.BlockSpec((NB, H), lambda i: (0, 0)),
        ],
        out_shape=[
            jax.ShapeDtypeStruct((NB, H), jnp.float32),
            jax.ShapeDtypeStruct((NB, H), jnp.float32),
        ],
    )(x, batch, w1, b1, w2row, b2)


def _poolB_call(x, batch, w1, b1, w2row, b2, gmax):
    npr = x.shape[0]
    nb = npr // RB

    def body(x_ref, bt_ref, w1_ref, b1_ref, w2_ref, b2_ref, gm_ref,
             se_ref, sx_ref):
        i = pl.program_id(0)
        xb = x_ref[...]
        g = _attg(xb, w1_ref[...], b1_ref[...], w2_ref[...], b2_ref[...])
        bt = bt_ref[...]
        ses, sxs = [], []
        for bb in range(NB):
            msk = bt == bb
            e = jnp.where(msk, jnp.exp(g - gm_ref[bb, 0]), 0.0)
            ses.append(jnp.sum(e, 0, keepdims=True))
            sxs.append(jnp.sum(e * xb, 0, keepdims=True))
        se = jnp.broadcast_to(jnp.concatenate(ses, 0), (NB, H))
        sx = jnp.concatenate(sxs, 0)

        @pl.when(i == 0)
        def _():
            se_ref[...] = se
            sx_ref[...] = sx

        @pl.when(i != 0)
        def _():
            se_ref[...] = se_ref[...] + se
            sx_ref[...] = sx_ref[...] + sx

    return pl.pallas_call(
        body,
        grid=(nb,),
        in_specs=[
            pl.BlockSpec((RB, H), lambda i: (i, 0)),
            pl.BlockSpec((RB, 1), lambda i: (i, 0)),
            pl.BlockSpec((H, H // 2), lambda i: (0, 0)),
            pl.BlockSpec((1, H // 2), lambda i: (0, 0)),
            pl.BlockSpec((1, H // 2), lambda i: (0, 0)),
            pl.BlockSpec((1, 1), lambda i: (0, 0)),
            pl.BlockSpec((NB, H), lambda i: (0, 0)),
        ],
        out_specs=[
            pl.BlockSpec((NB, H), lambda i: (0, 0)),
            pl.BlockSpec((NB, H), lambda i: (0, 0)),
        ],
        out_shape=[
            jax.ShapeDtypeStruct((NB, H), jnp.float32),
            jax.ShapeDtypeStruct((NB, H), jnp.float32),
        ],
    )(x, batch, w1, b1, w2row, b2, gmax)


def _head_call(vmax, cmax, vse, vsx, cse, csx, hp):
    def body(vm, cm, vse_r, vsx_r, cse_r, csx_r, w1, b1, w2, b2,
             wmu, bmu, wsg, bsg, o_ref):
        vatt = vsx_r[...] / (vse_r[...] + 1e-16)
        catt = csx_r[...] / (cse_r[...] + 1e-16)
        emb = jnp.concatenate([vm[...], cm[...], vatt, catt], axis=1)
        h = jnp.maximum(jnp.dot(emb, w1[...],
                                preferred_element_type=jnp.float32)
                        + b1[...], 0.0)
        h = jnp.maximum(jnp.dot(h, w2[...],
                                preferred_element_type=jnp.float32)
                        + b2[...], 0.0)
        mu = jnp.dot(h, wmu[...], preferred_element_type=jnp.float32) + bmu[...]
        sg = jnp.exp(jnp.dot(h, wsg[...], preferred_element_type=jnp.float32)
                     + bsg[...])
        o_ref[0] = mu
        o_ref[1] = sg

    out = pl.pallas_call(
        body,
        out_shape=jax.ShapeDtypeStruct((2, NB, CFG), jnp.float32),
    )(vmax, cmax, vse, vsx, cse, csx,
      hp['W1'], hp['b1'][None], hp['W2'], hp['b2'][None],
      hp['Wmu'], hp['bmu'][None], hp['Wsig'], hp['bsig'][None])
    return jnp.stack([out[0], out[1]], axis=-1)


def _pad_rows_cols(a, rows, cols):
    return jnp.pad(a, ((0, rows - a.shape[0]), (0, cols - a.shape[1])))


def kernel(var_feats, cstr_feats, edge_attr, edge_index, var_batch_el,
           cstr_batch_el, params):
    xv = _pad_rows_cols(var_feats, NVP, 16)
    xc = _pad_rows_cols(cstr_feats, NCP, 16)
    srcp = jnp.pad(edge_index[0], (0, EPAD - NEDGE),
                   constant_values=NCP - 1).reshape(EPAD // EB, EB)
    dstp = jnp.pad(edge_index[1], (0, EPAD - NEDGE),
                   constant_values=NCP - 1).reshape(EPAD // EB, EB)
    ewp = jnp.pad(edge_attr, (0, EPAD - NEDGE)).reshape(EPAD // EB, EB)

    deg_both = _deg_pass_call(dstp, srcp)
    degv2 = deg_both[:, 0]
    degc2 = deg_both[:, 1]

    for li, L in enumerate(params['layers']):
        d_v, d_c = xv.shape[1], xc.shape[1]
        if li == 0:
            ng = jnp.pad(L['nbn_g'], (0, 16 - 9), constant_values=1.0)[None]
            nb_ = jnp.pad(L['nbn_b'], (0, 16 - 9))[None]
            cg = jnp.pad(L['cbn_g'], (0, 16 - 1), constant_values=1.0)[None]
            cb_ = jnp.pad(L['cbn_b'], (0, 16 - 1))[None]
            nWroot = jnp.pad(L['nWroot'], ((0, 16 - 9), (0, 0)))
            cWroot = jnp.pad(L['cWroot'], ((0, 16 - 1), (0, 0)))
            nWrel = jnp.pad(L['nWrel'], ((0, 16 - 1), (0, 0)))
            cWrel = jnp.pad(L['cWrel'], ((0, 16 - 9), (0, 0)))
        else:
            ng, nb_ = L['nbn_g'][None], L['nbn_b'][None]
            cg, cb_ = L['cbn_g'][None], L['cbn_b'][None]
            nWroot, cWroot = L['nWroot'], L['cWroot']
            nWrel, cWrel = L['nWrel'], L['cWrel']

        stv = _stats_call(xv, d_v)
        stc = _stats_call(xc, d_c)
        xv_n, rootv = _norm_call(xv, stv, ng, nb_, nWroot, NVAR, d_v)
        xc_n, rootc = _norm_call(xc, stc, cg, cb_, cWroot, NCSTR, d_c)
        accv2 = _edge_pass_call(d_c, xc_n, srcp, dstp, ewp)
        accc2 = _edge_pass_call(d_v, xv_n, dstp, srcp, ewp)
        xv = _post_call(accv2, degv2, rootv, nWrel, L['nbrel'][None],
                        NVAR, d_c)
        xc = _post_call(accc2, degc2, rootc, cWrel, L['cbrel'][None],
                        NCSTR, d_v)

    vb = jnp.pad(var_batch_el, (0, NVP - NVAR), constant_values=NB)[:, None]
    cbt = jnp.pad(cstr_batch_el, (0, NCP - NCSTR), constant_values=NB)[:, None]
    av, ac = params['att_var'], params['att_cstr']
    aw2v = av['W2'][:, 0][None]
    aw2c = ac['W2'][:, 0][None]
    ab2v = av['b2'][None]
    ab2c = ac['b2'][None]
    vmax, vgm = _poolA_call(xv, vb, av['W1'], av['b1'][None], aw2v, ab2v)
    cmax, cgm = _poolA_call(xc, cbt, ac['W1'], ac['b1'][None], aw2c, ab2c)
    vse, vsx = _poolB_call(xv, vb, av['W1'], av['b1'][None], aw2v, ab2v, vgm)
    cse, csx = _poolB_call(xc, cbt, ac['W1'], ac['b1'][None], aw2c, ab2c, cgm)
    return _head_call(vmax, cmax, vse, vsx, cse, csx, params['head'])


# speedup vs baseline: 1.1494x; 1.0065x over previous
import functools

import jax
import jax.numpy as jnp
from jax import lax
from jax.experimental import pallas as pl
from jax.experimental.pallas import tpu as pltpu
from jax.experimental.pallas import tpu_sc as plsc

NVAR = 50000
NCSTR = 25000
NEDGE = 800000
NB = 16
H = 64
CFG = 32
NVP = 50176
NCP = 25088
EPAD = 819200
NC, NS, LANES = 2, 16, 16
NW = NC * NS
EB = 128
RB = 512
_NEG = -1e30


@functools.lru_cache(maxsize=None)
def _make_edge_pass(d):
    eb = 128 if d == 64 else 256
    per_w = EPAD // NW
    nblk = per_w // eb
    cnk = 10 if d == 64 else 20
    nchk = nblk // cnk
    rps = NCP // NS
    zrows = 49
    mesh = plsc.VectorSubcoreMesh(
        core_axis_name="c", subcore_axis_name="s",
        num_cores=NC, num_subcores=NS)

    def body(table, gidx, sidx, ew, out, gi_buf, si_buf, ew_buf, rows3,
             acc, gsem0, gsem1, gsem2, ssem0, ssem1, ssem2):
        c = lax.axis_index("c")
        s = lax.axis_index("s")
        gsems = (gsem0, gsem1, gsem2)
        ssems = (ssem0, ssem1, ssem2)
        zvec = jnp.zeros((LANES,), jnp.float32)

        def zrow(r, _):
            for kk in range(d // LANES):
                rows3[0, r, pl.ds(kk * LANES, LANES)] = zvec
            return 0
        lax.fori_loop(0, zrows, zrow, 0)
        zsrc = rows3.at[0].at[pl.ds(0, zrows)]

        def zcp(i, _):
            pltpu.sync_copy(zsrc, acc.at[pl.ds(s * rps + i * zrows, zrows)])
            return 0
        lax.fori_loop(0, rps // zrows, zcp, 0)

        rbase = (c * NS + s) * nblk
        plsc.subcore_barrier()

        def start_gather(b, j):
            pltpu.async_copy(table.at[gi_buf.at[j]], rows3.at[b], gsems[b])

        def wait_gather(b, j):
            pltpu.make_async_copy(table.at[gi_buf.at[j]], rows3.at[b],
                                  gsems[b]).wait()

        def start_scatter(b, j):
            pltpu.async_copy(rows3.at[b], acc.at[si_buf.at[j]], ssems[b],
                             add=True)

        def wait_scatter(b, j):
            pltpu.make_async_copy(rows3.at[b], acc.at[si_buf.at[j]],
                                  ssems[b]).wait()

        def scale(b, j):
            def esc(q, _):
                jb = lax.broadcast(j, (LANES,))
                for u in range(4):
                    e = 4 * q + u
                    evec = lax.broadcast(e, (LANES,))
                    ewb = plsc.load_gather(ew_buf, [jb, evec])
                    for kk in range(d // LANES):
                        sl = pl.ds(kk * LANES, LANES)
                        rows3[b, e, sl] = rows3[b, e, sl] * ewb
                return 0
            lax.fori_loop(0, eb // 4, esc, 0)

        def chunk(ci, _):
            crow = rbase + ci * cnk
            pltpu.sync_copy(gidx.at[pl.ds(crow, cnk)], gi_buf)
            pltpu.sync_copy(sidx.at[pl.ds(crow, cnk)], si_buf)
            pltpu.sync_copy(ew.at[pl.ds(crow, cnk)], ew_buf)
            start_gather(0, 0)
            for j in range(cnk):
                b = j % 3
                nb_buf = (j + 1) % 3
                if j >= 2:
                    wait_scatter(nb_buf, j - 2)
                if j + 1 < cnk:
                    start_gather(nb_buf, j + 1)
                wait_gather(b, j)
                scale(b, j)
                start_scatter(b, j)
            wait_scatter((cnk - 2) % 3, cnk - 2)
            wait_scatter((cnk - 1) % 3, cnk - 1)
            return 0
        lax.fori_loop(0, nchk, chunk, 0)
        plsc.subcore_barrier()
        pltpu.sync_copy(acc.at[pl.ds(s * rps, rps)],
                        out.at[c].at[pl.ds(s * rps, rps)])

    return pl.kernel(
        body,
        out_type=jax.ShapeDtypeStruct((NC, NCP, d), jnp.float32),
        mesh=mesh,
        compiler_params=pltpu.CompilerParams(
            needs_layout_passes=False, use_tc_tiling_on_sc=False),
        scratch_types=[
            pltpu.VMEM((cnk, eb), jnp.int32),
            pltpu.VMEM((cnk, eb), jnp.int32),
            pltpu.VMEM((cnk, eb), jnp.float32),
            pltpu.VMEM((3, eb, d), jnp.float32),
            pltpu.VMEM_SHARED((NCP, d), jnp.float32),
            pltpu.SemaphoreType.DMA,
            pltpu.SemaphoreType.DMA,
            pltpu.SemaphoreType.DMA,
            pltpu.SemaphoreType.DMA,
            pltpu.SemaphoreType.DMA,
            pltpu.SemaphoreType.DMA,
        ],
    )


def _edge_pass_call(d, table, gidx, sidx, ew):
    eb = 128 if d == 64 else 256
    return _make_edge_pass(d)(table, gidx.reshape(EPAD // eb, eb),
                              sidx.reshape(EPAD // eb, eb),
                              ew.reshape(EPAD // eb, eb))


@functools.lru_cache(maxsize=None)
def _make_deg_pass():
    d = 16
    per_w = EPAD // NW
    nblk = per_w // EB
    cnk = 25
    nchk = nblk // cnk
    rps = NCP // NS
    zrows = 49
    mesh = plsc.VectorSubcoreMesh(
        core_axis_name="c", subcore_axis_name="s",
        num_cores=NC, num_subcores=NS)

    def body(sidx_a, sidx_b, out, ia_buf, ib_buf, ones_v, acc2, sem_a, sem_b):
        c = lax.axis_index("c")
        s = lax.axis_index("s")
        onev = jnp.ones((LANES,), jnp.float32)
        zvec = jnp.zeros((LANES,), jnp.float32)

        def orow(r, _):
            ones_v[r, pl.ds(0, LANES)] = onev
            return 0
        lax.fori_loop(0, EB, orow, 0)

        def zrow(r, _):
            ones_v[EB + r, pl.ds(0, LANES)] = zvec
            return 0
        lax.fori_loop(0, zrows, zrow, 0)
        zsrc = ones_v.at[pl.ds(EB, zrows)]

        def zcp(i, _):
            pltpu.sync_copy(zsrc,
                            acc2.at[0].at[pl.ds(s * rps + i * zrows, zrows)])
            pltpu.sync_copy(zsrc,
                            acc2.at[1].at[pl.ds(s * rps + i * zrows, zrows)])
            return 0
        lax.fori_loop(0, rps // zrows, zcp, 0)

        rbase = (c * NS + s) * nblk
        plsc.subcore_barrier()
        ones_rows = ones_v.at[pl.ds(0, EB)]

        def chunk(ci, _):
            crow = rbase + ci * cnk
            pltpu.sync_copy(sidx_a.at[pl.ds(crow, cnk)], ia_buf)
            pltpu.sync_copy(sidx_b.at[pl.ds(crow, cnk)], ib_buf)
            for j in range(cnk):
                if j >= 1:
                    pltpu.make_async_copy(
                        ones_rows, acc2.at[0].at[ia_buf.at[j - 1]],
                        sem_a).wait()
                    pltpu.make_async_copy(
                        ones_rows, acc2.at[1].at[ib_buf.at[j - 1]],
                        sem_b).wait()
                pltpu.async_copy(ones_rows, acc2.at[0].at[ia_buf.at[j]],
                                 sem_a, add=True)
                pltpu.async_copy(ones_rows, acc2.at[1].at[ib_buf.at[j]],
                                 sem_b, add=True)
            pltpu.make_async_copy(ones_rows, acc2.at[0].at[ia_buf.at[cnk - 1]],
                                  sem_a).wait()
            pltpu.make_async_copy(ones_rows, acc2.at[1].at[ib_buf.at[cnk - 1]],
                                  sem_b).wait()
            return 0
        lax.fori_loop(0, nchk, chunk, 0)
        plsc.subcore_barrier()
        pltpu.sync_copy(acc2.at[0].at[pl.ds(s * rps, rps)],
                        out.at[c].at[0].at[pl.ds(s * rps, rps)])
        pltpu.sync_copy(acc2.at[1].at[pl.ds(s * rps, rps)],
                        out.at[c].at[1].at[pl.ds(s * rps, rps)])

    return pl.kernel(
        body,
        out_type=jax.ShapeDtypeStruct((NC, 2, NCP, d), jnp.float32),
        mesh=mesh,
        compiler_params=pltpu.CompilerParams(
            needs_layout_passes=False, use_tc_tiling_on_sc=False),
        scratch_types=[
            pltpu.VMEM((cnk, EB), jnp.int32),
            pltpu.VMEM((cnk, EB), jnp.int32),
            pltpu.VMEM((EB + zrows, d), jnp.float32),
            pltpu.VMEM_SHARED((2, NCP, d), jnp.float32),
            pltpu.SemaphoreType.DMA,
            pltpu.SemaphoreType.DMA,
        ],
    )


def _deg_pass_call(dstp, srcp):
    return _make_deg_pass()(dstp, srcp)


def _stats_call(x, d):
    nb = x.shape[0] // RB

    def body(x_ref, o_ref):
        i = pl.program_id(0)
        blk = x_ref[...]
        s = jnp.sum(blk, axis=0, keepdims=True)
        sq = jnp.sum(blk * blk, axis=0, keepdims=True)
        part = jnp.concatenate([s, sq, jnp.zeros((6, d), jnp.float32)], 0)

        @pl.when(i == 0)
        def _():
            o_ref[...] = part

        @pl.when(i != 0)
        def _():
            o_ref[...] = o_ref[...] + part

    return pl.pallas_call(
        body,
        grid=(nb,),
        in_specs=[pl.BlockSpec((RB, d), lambda i: (i, 0))],
        out_specs=pl.BlockSpec((8, d), lambda i: (0, 0)),
        out_shape=jax.ShapeDtypeStruct((8, d), jnp.float32),
    )(x)


def _norm_call(x, stats, g, b, wroot, n_true, d):
    npr = x.shape[0]
    nb = npr // RB
    inv_n = 1.0 / n_true

    def body(x_ref, st_ref, g_ref, b_ref, w_ref, xn_ref, rt_ref):
        i = pl.program_id(0)
        m = st_ref[0:1, :] * inv_n
        v = st_ref[1:2, :] * inv_n - m * m
        rs = lax.rsqrt(v + 1e-5)
        xn = (x_ref[...] - m) * rs * g_ref[...] + b_ref[...]
        rid = i * RB + lax.broadcasted_iota(jnp.int32, (RB, 1), 0)
        xn = jnp.where(rid < n_true, xn, 0.0)
        xn_ref[...] = xn
        rt_ref[...] = jnp.dot(xn, w_ref[...],
                              preferred_element_type=jnp.float32)

    return pl.pallas_call(
        body,
        grid=(nb,),
        in_specs=[
            pl.BlockSpec((RB, d), lambda i: (i, 0)),
            pl.BlockSpec((8, d), lambda i: (0, 0)),
            pl.BlockSpec((1, d), lambda i: (0, 0)),
            pl.BlockSpec((1, d), lambda i: (0, 0)),
            pl.BlockSpec((d, H), lambda i: (0, 0)),
        ],
        out_specs=[
            pl.BlockSpec((RB, d), lambda i: (i, 0)),
            pl.BlockSpec((RB, H), lambda i: (i, 0)),
        ],
        out_shape=[
            jax.ShapeDtypeStruct((npr, d), jnp.float32),
            jax.ShapeDtypeStruct((npr, H), jnp.float32),
        ],
    )(x, stats, g, b, wroot)


def _post_call(acc2, deg2, root, wrel, brel, n_true, d):
    npr = root.shape[0]
    nb = npr // RB
    nacc = NCP // RB

    def body(a0, a1, d0, d1, rt_ref, w_ref, bb_ref, y_ref):
        i = pl.program_id(0)
        valid = (i < nacc).astype(jnp.float32)
        agg = (a0[0] + a1[0]) * valid
        dg = (d0[0][:, 0:1] + d1[0][:, 0:1]) * valid
        agg = agg / jnp.maximum(dg, 1.0)
        y = (jnp.dot(agg, w_ref[...], preferred_element_type=jnp.float32)
             + bb_ref[...] + rt_ref[...])
        rid = i * RB + lax.broadcasted_iota(jnp.int32, (RB, 1), 0)
        y_ref[...] = jnp.where(rid < n_true, jnp.maximum(y, 0.0), 0.0)

    amap = lambda p: (lambda i: (p, jnp.minimum(i, nacc - 1), 0))
    return pl.pallas_call(
        body,
        grid=(nb,),
        in_specs=[
            pl.BlockSpec((1, RB, d), amap(0)),
            pl.BlockSpec((1, RB, d), amap(1)),
            pl.BlockSpec((1, RB, 16), amap(0)),
            pl.BlockSpec((1, RB, 16), amap(1)),
            pl.BlockSpec((RB, H), lambda i: (i, 0)),
            pl.BlockSpec((d, H), lambda i: (0, 0)),
            pl.BlockSpec((1, H), lambda i: (0, 0)),
        ],
        out_specs=pl.BlockSpec((RB, H), lambda i: (i, 0)),
        out_shape=jax.ShapeDtypeStruct((npr, H), jnp.float32),
    )(acc2, acc2, deg2, deg2, root, wrel, brel)


def _attg(xb, w1, b1, w2row, b2):
    h = jnp.maximum(jnp.dot(xb, w1, preferred_element_type=jnp.float32)
                    + b1, 0.0)
    return jnp.sum(h * w2row, axis=1, keepdims=True) + b2


def _poolA_call(x, batch, w1, b1, w2row, b2):
    npr = x.shape[0]
    nb = npr // RB

    def body(x_ref, bt_ref, w1_ref, b1_ref, w2_ref, b2_ref, xm_ref, gm_ref):
        i = pl.program_id(0)
        xb = x_ref[...]
        g = _attg(xb, w1_ref[...], b1_ref[...], w2_ref[...], b2_ref[...])
        gb = jnp.broadcast_to(g, (RB, H))
        bt = bt_ref[...]
        xms, gms = [], []
        for bb in range(NB):
            msk = bt == bb
            xms.append(jnp.max(jnp.where(msk, xb, _NEG), 0, keepdims=True))
            gms.append(jnp.max(jnp.where(msk, gb, _NEG), 0, keepdims=True))
        xm = jnp.concatenate(xms, 0)
        gm = jnp.concatenate(gms, 0)

        @pl.when(i == 0)
        def _():
            xm_ref[...] = xm
            gm_ref[...] = gm

        @pl.when(i != 0)
        def _():
            xm_ref[...] = jnp.maximum(xm_ref[...], xm)
            gm_ref[...] = jnp.maximum(gm_ref[...], gm)

    return pl.pallas_call(
        body,
        grid=(nb,),
        in_specs=[
            pl.BlockSpec((RB, H), lambda i: (i, 0)),
            pl.BlockSpec((RB, 1), lambda i: (i, 0)),
            pl.BlockSpec((H, H // 2), lambda i: (0, 0)),
            pl.BlockSpec((1, H // 2), lambda i: (0, 0)),
            pl.BlockSpec((1, H // 2), lambda i: (0, 0)),
            pl.BlockSpec((1, 1), lambda i: (0, 0)),
        ],
        out_specs=[
            pl.BlockSpec((NB, H), lambda i: (0, 0)),
            pl.BlockSpec((NB, H), lambda i: (0, 0)),
        ],
        out_shape=[
            jax.ShapeDtypeStruct((NB, H), jnp.float32),
            jax.ShapeDtypeStruct((NB, H), jnp.float32),
        ],
    )(x, batch, w1, b1, w2row, b2)


def _poolB_call(x, batch, w1, b1, w2row, b2, gmax):
    npr = x.shape[0]
    nb = npr // RB

    def body(x_ref, bt_ref, w1_ref, b1_ref, w2_ref, b2_ref, gm_ref,
             se_ref, sx_ref):
        i = pl.program_id(0)
        xb = x_ref[...]
        g = _attg(xb, w1_ref[...], b1_ref[...], w2_ref[...], b2_ref[...])
        bt = bt_ref[...]
        ses, sxs = [], []
        for bb in range(NB):
            msk = bt == bb
            e = jnp.where(msk, jnp.exp(g - gm_ref[bb, 0]), 0.0)
            ses.append(jnp.sum(e, 0, keepdims=True))
            sxs.append(jnp.sum(e * xb, 0, keepdims=True))
        se = jnp.broadcast_to(jnp.concatenate(ses, 0), (NB, H))
        sx = jnp.concatenate(sxs, 0)

        @pl.when(i == 0)
        def _():
            se_ref[...] = se
            sx_ref[...] = sx

        @pl.when(i != 0)
        def _():
            se_ref[...] = se_ref[...] + se
            sx_ref[...] = sx_ref[...] + sx

    return pl.pallas_call(
        body,
        grid=(nb,),
        in_specs=[
            pl.BlockSpec((RB, H), lambda i: (i, 0)),
            pl.BlockSpec((RB, 1), lambda i: (i, 0)),
            pl.BlockSpec((H, H // 2), lambda i: (0, 0)),
            pl.BlockSpec((1, H // 2), lambda i: (0, 0)),
            pl.BlockSpec((1, H // 2), lambda i: (0, 0)),
            pl.BlockSpec((1, 1), lambda i: (0, 0)),
            pl.BlockSpec((NB, H), lambda i: (0, 0)),
        ],
        out_specs=[
            pl.BlockSpec((NB, H), lambda i: (0, 0)),
            pl.BlockSpec((NB, H), lambda i: (0, 0)),
        ],
        out_shape=[
            jax.ShapeDtypeStruct((NB, H), jnp.float32),
            jax.ShapeDtypeStruct((NB, H), jnp.float32),
        ],
    )(x, batch, w1, b1, w2row, b2, gmax)


def _head_call(vmax, cmax, vse, vsx, cse, csx, hp):
    def body(vm, cm, vse_r, vsx_r, cse_r, csx_r, w1, b1, w2, b2,
             wmu, bmu, wsg, bsg, o_ref):
        vatt = vsx_r[...] / (vse_r[...] + 1e-16)
        catt = csx_r[...] / (cse_r[...] + 1e-16)
        emb = jnp.concatenate([vm[...], cm[...], vatt, catt], axis=1)
        h = jnp.maximum(jnp.dot(emb, w1[...],
                                preferred_element_type=jnp.float32)
                        + b1[...], 0.0)
        h = jnp.maximum(jnp.dot(h, w2[...],
                                preferred_element_type=jnp.float32)
                        + b2[...], 0.0)
        mu = jnp.dot(h, wmu[...], preferred_element_type=jnp.float32) + bmu[...]
        sg = jnp.exp(jnp.dot(h, wsg[...], preferred_element_type=jnp.float32)
                     + bsg[...])
        o_ref[0] = mu
        o_ref[1] = sg

    out = pl.pallas_call(
        body,
        out_shape=jax.ShapeDtypeStruct((2, NB, CFG), jnp.float32),
    )(vmax, cmax, vse, vsx, cse, csx,
      hp['W1'], hp['b1'][None], hp['W2'], hp['b2'][None],
      hp['Wmu'], hp['bmu'][None], hp['Wsig'], hp['bsig'][None])
    return jnp.stack([out[0], out[1]], axis=-1)


def _pad_rows_cols(a, rows, cols):
    return jnp.pad(a, ((0, rows - a.shape[0]), (0, cols - a.shape[1])))


def kernel(var_feats, cstr_feats, edge_attr, edge_index, var_batch_el,
           cstr_batch_el, params):
    xv = _pad_rows_cols(var_feats, NVP, 16)
    xc = _pad_rows_cols(cstr_feats, NCP, 16)
    srcp = jnp.pad(edge_index[0], (0, EPAD - NEDGE),
                   constant_values=NCP - 1).reshape(EPAD // EB, EB)
    dstp = jnp.pad(edge_index[1], (0, EPAD - NEDGE),
                   constant_values=NCP - 1).reshape(EPAD // EB, EB)
    ewp = jnp.pad(edge_attr, (0, EPAD - NEDGE)).reshape(EPAD // EB, EB)

    deg_both = _deg_pass_call(dstp, srcp)
    degv2 = deg_both[:, 0]
    degc2 = deg_both[:, 1]

    for li, L in enumerate(params['layers']):
        d_v, d_c = xv.shape[1], xc.shape[1]
        if li == 0:
            ng = jnp.pad(L['nbn_g'], (0, 16 - 9), constant_values=1.0)[None]
            nb_ = jnp.pad(L['nbn_b'], (0, 16 - 9))[None]
            cg = jnp.pad(L['cbn_g'], (0, 16 - 1), constant_values=1.0)[None]
            cb_ = jnp.pad(L['cbn_b'], (0, 16 - 1))[None]
            nWroot = jnp.pad(L['nWroot'], ((0, 16 - 9), (0, 0)))
            cWroot = jnp.pad(L['cWroot'], ((0, 16 - 1), (0, 0)))
            nWrel = jnp.pad(L['nWrel'], ((0, 16 - 1), (0, 0)))
            cWrel = jnp.pad(L['cWrel'], ((0, 16 - 9), (0, 0)))
        else:
            ng, nb_ = L['nbn_g'][None], L['nbn_b'][None]
            cg, cb_ = L['cbn_g'][None], L['cbn_b'][None]
            nWroot, cWroot = L['nWroot'], L['cWroot']
            nWrel, cWrel = L['nWrel'], L['cWrel']

        stv = _stats_call(xv, d_v)
        stc = _stats_call(xc, d_c)
        xv_n, rootv = _norm_call(xv, stv, ng, nb_, nWroot, NVAR, d_v)
        xc_n, rootc = _norm_call(xc, stc, cg, cb_, cWroot, NCSTR, d_c)
        accv2 = _edge_pass_call(d_c, xc_n, srcp, dstp, ewp)
        accc2 = _edge_pass_call(d_v, xv_n, dstp, srcp, ewp)
        xv = _post_call(accv2, degv2, rootv, nWrel, L['nbrel'][None],
                        NVAR, d_c)
        xc = _post_call(accc2, degc2, rootc, cWrel, L['cbrel'][None],
                        NCSTR, d_v)

    vb = jnp.pad(var_batch_el, (0, NVP - NVAR), constant_values=NB)[:, None]
    cbt = jnp.pad(cstr_batch_el, (0, NCP - NCSTR), constant_values=NB)[:, None]
    av, ac = params['att_var'], params['att_cstr']
    aw2v = av['W2'][:, 0][None]
    aw2c = ac['W2'][:, 0][None]
    ab2v = av['b2'][None]
    ab2c = ac['b2'][None]
    vmax, vgm = _poolA_call(xv, vb, av['W1'], av['b1'][None], aw2v, ab2v)
    cmax, cgm = _poolA_call(xc, cbt, ac['W1'], ac['b1'][None], aw2c, ab2c)
    vse, vsx = _poolB_call(xv, vb, av['W1'], av['b1'][None], aw2v, ab2v, vgm)
    cse, csx = _poolB_call(xc, cbt, ac['W1'], ac['b1'][None], aw2c, ab2c, cgm)
    return _head_call(vmax, cmax, vse, vsx, cse, csx, params['head'])


# speedup vs baseline: 1.1495x; 1.0001x over previous
import functools

import jax
import jax.numpy as jnp
from jax import lax
from jax.experimental import pallas as pl
from jax.experimental.pallas import tpu as pltpu
from jax.experimental.pallas import tpu_sc as plsc

NVAR = 50000
NCSTR = 25000
NEDGE = 800000
NB = 16
H = 64
CFG = 32
NVP = 50176
NCP = 25088
EPAD = 819200
NC, NS, LANES = 2, 16, 16
NW = NC * NS
EB = 128
RB = 512
_NEG = -1e30


@functools.lru_cache(maxsize=None)
def _make_edge_pass(d):
    eb = 128 if d == 64 else 256
    per_w = EPAD // NW
    nblk = per_w // eb
    cnk = 10 if d == 64 else 20
    nchk = nblk // cnk
    rps = NCP // NS
    zrows = 49
    mesh = plsc.VectorSubcoreMesh(
        core_axis_name="c", subcore_axis_name="s",
        num_cores=NC, num_subcores=NS)

    def body(table, gidx, sidx, ew, out, gi_buf, si_buf, ew_buf, rows3,
             acc, gsem0, gsem1, gsem2, ssem0, ssem1, ssem2):
        c = lax.axis_index("c")
        s = lax.axis_index("s")
        gsems = (gsem0, gsem1, gsem2)
        ssems = (ssem0, ssem1, ssem2)
        zvec = jnp.zeros((LANES,), jnp.float32)

        def zrow(r, _):
            for kk in range(d // LANES):
                rows3[0, r, pl.ds(kk * LANES, LANES)] = zvec
            return 0
        lax.fori_loop(0, zrows, zrow, 0)
        zsrc = rows3.at[0].at[pl.ds(0, zrows)]

        def zcp(i, _):
            pltpu.sync_copy(zsrc, acc.at[pl.ds(s * rps + i * zrows, zrows)])
            return 0
        lax.fori_loop(0, rps // zrows, zcp, 0)

        rbase = (c * NS + s) * nblk
        plsc.subcore_barrier()

        def start_gather(b, j):
            pltpu.async_copy(table.at[gi_buf.at[j]], rows3.at[b], gsems[b])

        def wait_gather(b, j):
            pltpu.make_async_copy(table.at[gi_buf.at[j]], rows3.at[b],
                                  gsems[b]).wait()

        def start_scatter(b, j):
            pltpu.async_copy(rows3.at[b], acc.at[si_buf.at[j]], ssems[b],
                             add=True)

        def wait_scatter(b, j):
            pltpu.make_async_copy(rows3.at[b], acc.at[si_buf.at[j]],
                                  ssems[b]).wait()

        def scale(b, j):
            def esc(q, _):
                jb = lax.broadcast(j, (LANES,))
                for u in range(4):
                    e = 4 * q + u
                    evec = lax.broadcast(e, (LANES,))
                    ewb = plsc.load_gather(ew_buf, [jb, evec])
                    for kk in range(d // LANES):
                        sl = pl.ds(kk * LANES, LANES)
                        rows3[b, e, sl] = rows3[b, e, sl] * ewb
                return 0
            lax.fori_loop(0, eb // 4, esc, 0)

        def chunk(ci, _):
            crow = rbase + ci * cnk
            pltpu.sync_copy(gidx.at[pl.ds(crow, cnk)], gi_buf)
            pltpu.sync_copy(sidx.at[pl.ds(crow, cnk)], si_buf)
            pltpu.sync_copy(ew.at[pl.ds(crow, cnk)], ew_buf)
            start_gather(0, 0)
            for j in range(cnk):
                b = j % 3
                nb_buf = (j + 1) % 3
                if j >= 2:
                    wait_scatter(nb_buf, j - 2)
                if j + 1 < cnk:
                    start_gather(nb_buf, j + 1)
                wait_gather(b, j)
                scale(b, j)
                start_scatter(b, j)
            wait_scatter((cnk - 2) % 3, cnk - 2)
            wait_scatter((cnk - 1) % 3, cnk - 1)
            return 0
        lax.fori_loop(0, nchk, chunk, 0)
        plsc.subcore_barrier()
        pltpu.sync_copy(acc.at[pl.ds(s * rps, rps)],
                        out.at[c].at[pl.ds(s * rps, rps)])

    return pl.kernel(
        body,
        out_type=jax.ShapeDtypeStruct((NC, NCP, d), jnp.float32),
        mesh=mesh,
        compiler_params=pltpu.CompilerParams(
            needs_layout_passes=False, use_tc_tiling_on_sc=False),
        scratch_types=[
            pltpu.VMEM((cnk, eb), jnp.int32),
            pltpu.VMEM((cnk, eb), jnp.int32),
            pltpu.VMEM((cnk, eb), jnp.float32),
            pltpu.VMEM((3, eb, d), jnp.float32),
            pltpu.VMEM_SHARED((NCP, d), jnp.float32),
            pltpu.SemaphoreType.DMA,
            pltpu.SemaphoreType.DMA,
            pltpu.SemaphoreType.DMA,
            pltpu.SemaphoreType.DMA,
            pltpu.SemaphoreType.DMA,
            pltpu.SemaphoreType.DMA,
        ],
    )


def _edge_pass_call(d, table, gidx, sidx, ew):
    eb = 128 if d == 64 else 256
    return _make_edge_pass(d)(table, gidx.reshape(EPAD // eb, eb),
                              sidx.reshape(EPAD // eb, eb),
                              ew.reshape(EPAD // eb, eb))


@functools.lru_cache(maxsize=None)
def _make_deg_pass():
    d = 16
    per_w = EPAD // NW
    eb = 256
    nblk = per_w // eb
    cnk = 20
    nchk = nblk // cnk
    rps = NCP // NS
    zrows = 49
    mesh = plsc.VectorSubcoreMesh(
        core_axis_name="c", subcore_axis_name="s",
        num_cores=NC, num_subcores=NS)

    def body(sidx_a, sidx_b, out, ia_buf, ib_buf, ones_v, acc2, sem_a, sem_b):
        c = lax.axis_index("c")
        s = lax.axis_index("s")
        onev = jnp.ones((LANES,), jnp.float32)
        zvec = jnp.zeros((LANES,), jnp.float32)

        def orow(r, _):
            ones_v[r, pl.ds(0, LANES)] = onev
            return 0
        lax.fori_loop(0, eb, orow, 0)

        def zrow(r, _):
            ones_v[eb + r, pl.ds(0, LANES)] = zvec
            return 0
        lax.fori_loop(0, zrows, zrow, 0)
        zsrc = ones_v.at[pl.ds(eb, zrows)]

        def zcp(i, _):
            pltpu.sync_copy(zsrc,
                            acc2.at[0].at[pl.ds(s * rps + i * zrows, zrows)])
            pltpu.sync_copy(zsrc,
                            acc2.at[1].at[pl.ds(s * rps + i * zrows, zrows)])
            return 0
        lax.fori_loop(0, rps // zrows, zcp, 0)

        rbase = (c * NS + s) * nblk
        plsc.subcore_barrier()
        ones_rows = ones_v.at[pl.ds(0, eb)]

        def chunk(ci, _):
            crow = rbase + ci * cnk
            pltpu.sync_copy(sidx_a.at[pl.ds(crow, cnk)], ia_buf)
            pltpu.sync_copy(sidx_b.at[pl.ds(crow, cnk)], ib_buf)
            for j in range(cnk):
                if j >= 1:
                    pltpu.make_async_copy(
                        ones_rows, acc2.at[0].at[ia_buf.at[j - 1]],
                        sem_a).wait()
                    pltpu.make_async_copy(
                        ones_rows, acc2.at[1].at[ib_buf.at[j - 1]],
                        sem_b).wait()
                pltpu.async_copy(ones_rows, acc2.at[0].at[ia_buf.at[j]],
                                 sem_a, add=True)
                pltpu.async_copy(ones_rows, acc2.at[1].at[ib_buf.at[j]],
                                 sem_b, add=True)
            pltpu.make_async_copy(ones_rows, acc2.at[0].at[ia_buf.at[cnk - 1]],
                                  sem_a).wait()
            pltpu.make_async_copy(ones_rows, acc2.at[1].at[ib_buf.at[cnk - 1]],
                                  sem_b).wait()
            return 0
        lax.fori_loop(0, nchk, chunk, 0)
        plsc.subcore_barrier()
        pltpu.sync_copy(acc2.at[0].at[pl.ds(s * rps, rps)],
                        out.at[c].at[0].at[pl.ds(s * rps, rps)])
        pltpu.sync_copy(acc2.at[1].at[pl.ds(s * rps, rps)],
                        out.at[c].at[1].at[pl.ds(s * rps, rps)])

    return pl.kernel(
        body,
        out_type=jax.ShapeDtypeStruct((NC, 2, NCP, d), jnp.float32),
        mesh=mesh,
        compiler_params=pltpu.CompilerParams(
            needs_layout_passes=False, use_tc_tiling_on_sc=False),
        scratch_types=[
            pltpu.VMEM((cnk, eb), jnp.int32),
            pltpu.VMEM((cnk, eb), jnp.int32),
            pltpu.VMEM((eb + zrows, d), jnp.float32),
            pltpu.VMEM_SHARED((2, NCP, d), jnp.float32),
            pltpu.SemaphoreType.DMA,
            pltpu.SemaphoreType.DMA,
        ],
    )


def _deg_pass_call(dstp, srcp):
    return _make_deg_pass()(dstp.reshape(EPAD // 256, 256),
                            srcp.reshape(EPAD // 256, 256))


def _stats_call(x, d):
    nb = x.shape[0] // RB

    def body(x_ref, o_ref):
        i = pl.program_id(0)
        blk = x_ref[...]
        s = jnp.sum(blk, axis=0, keepdims=True)
        sq = jnp.sum(blk * blk, axis=0, keepdims=True)
        part = jnp.concatenate([s, sq, jnp.zeros((6, d), jnp.float32)], 0)

        @pl.when(i == 0)
        def _():
            o_ref[...] = part

        @pl.when(i != 0)
        def _():
            o_ref[...] = o_ref[...] + part

    return pl.pallas_call(
        body,
        grid=(nb,),
        in_specs=[pl.BlockSpec((RB, d), lambda i: (i, 0))],
        out_specs=pl.BlockSpec((8, d), lambda i: (0, 0)),
        out_shape=jax.ShapeDtypeStruct((8, d), jnp.float32),
    )(x)


def _norm_call(x, stats, g, b, wroot, n_true, d):
    npr = x.shape[0]
    nb = npr // RB
    inv_n = 1.0 / n_true

    def body(x_ref, st_ref, g_ref, b_ref, w_ref, xn_ref, rt_ref):
        i = pl.program_id(0)
        m = st_ref[0:1, :] * inv_n
        v = st_ref[1:2, :] * inv_n - m * m
        rs = lax.rsqrt(v + 1e-5)
        xn = (x_ref[...] - m) * rs * g_ref[...] + b_ref[...]
        rid = i * RB + lax.broadcasted_iota(jnp.int32, (RB, 1), 0)
        xn = jnp.where(rid < n_true, xn, 0.0)
        xn_ref[...] = xn
        rt_ref[...] = jnp.dot(xn, w_ref[...],
                              preferred_element_type=jnp.float32)

    return pl.pallas_call(
        body,
        grid=(nb,),
        in_specs=[
            pl.BlockSpec((RB, d), lambda i: (i, 0)),
            pl.BlockSpec((8, d), lambda i: (0, 0)),
            pl.BlockSpec((1, d), lambda i: (0, 0)),
            pl.BlockSpec((1, d), lambda i: (0, 0)),
            pl.BlockSpec((d, H), lambda i: (0, 0)),
        ],
        out_specs=[
            pl.BlockSpec((RB, d), lambda i: (i, 0)),
            pl.BlockSpec((RB, H), lambda i: (i, 0)),
        ],
        out_shape=[
            jax.ShapeDtypeStruct((npr, d), jnp.float32),
            jax.ShapeDtypeStruct((npr, H), jnp.float32),
        ],
    )(x, stats, g, b, wroot)


def _post_call(acc2, deg2, root, wrel, brel, n_true, d):
    npr = root.shape[0]
    nb = npr // RB
    nacc = NCP // RB

    def body(a0, a1, d0, d1, rt_ref, w_ref, bb_ref, y_ref):
        i = pl.program_id(0)
        valid = (i < nacc).astype(jnp.float32)
        agg = (a0[0] + a1[0]) * valid
        dg = (d0[0][:, 0:1] + d1[0][:, 0:1]) * valid
        agg = agg / jnp.maximum(dg, 1.0)
        y = (jnp.dot(agg, w_ref[...], preferred_element_type=jnp.float32)
             + bb_ref[...] + rt_ref[...])
        rid = i * RB + lax.broadcasted_iota(jnp.int32, (RB, 1), 0)
        y_ref[...] = jnp.where(rid < n_true, jnp.maximum(y, 0.0), 0.0)

    amap = lambda p: (lambda i: (p, jnp.minimum(i, nacc - 1), 0))
    return pl.pallas_call(
        body,
        grid=(nb,),
        in_specs=[
            pl.BlockSpec((1, RB, d), amap(0)),
            pl.BlockSpec((1, RB, d), amap(1)),
            pl.BlockSpec((1, RB, 16), amap(0)),
            pl.BlockSpec((1, RB, 16), amap(1)),
            pl.BlockSpec((RB, H), lambda i: (i, 0)),
            pl.BlockSpec((d, H), lambda i: (0, 0)),
            pl.BlockSpec((1, H), lambda i: (0, 0)),
        ],
        out_specs=pl.BlockSpec((RB, H), lambda i: (i, 0)),
        out_shape=jax.ShapeDtypeStruct((npr, H), jnp.float32),
    )(acc2, acc2, deg2, deg2, root, wrel, brel)


def _attg(xb, w1, b1, w2row, b2):
    h = jnp.maximum(jnp.dot(xb, w1, preferred_element_type=jnp.float32)
                    + b1, 0.0)
    return jnp.sum(h * w2row, axis=1, keepdims=True) + b2


def _poolA_call(x, batch, w1, b1, w2row, b2):
    npr = x.shape[0]
    nb = npr // RB

    def body(x_ref, bt_ref, w1_ref, b1_ref, w2_ref, b2_ref, xm_ref, gm_ref):
        i = pl.program_id(0)
        xb = x_ref[...]
        g = _attg(xb, w1_ref[...], b1_ref[...], w2_ref[...], b2_ref[...])
        gb = jnp.broadcast_to(g, (RB, H))
        bt = bt_ref[...]
        xms, gms = [], []
        for bb in range(NB):
            msk = bt == bb
            xms.append(jnp.max(jnp.where(msk, xb, _NEG), 0, keepdims=True))
            gms.append(jnp.max(jnp.where(msk, gb, _NEG), 0, keepdims=True))
        xm = jnp.concatenate(xms, 0)
        gm = jnp.concatenate(gms, 0)

        @pl.when(i == 0)
        def _():
            xm_ref[...] = xm
            gm_ref[...] = gm

        @pl.when(i != 0)
        def _():
            xm_ref[...] = jnp.maximum(xm_ref[...], xm)
            gm_ref[...] = jnp.maximum(gm_ref[...], gm)

    return pl.pallas_call(
        body,
        grid=(nb,),
        in_specs=[
            pl.BlockSpec((RB, H), lambda i: (i, 0)),
            pl.BlockSpec((RB, 1), lambda i: (i, 0)),
            pl.BlockSpec((H, H // 2), lambda i: (0, 0)),
            pl.BlockSpec((1, H // 2), lambda i: (0, 0)),
            pl.BlockSpec((1, H // 2), lambda i: (0, 0)),
            pl.BlockSpec((1, 1), lambda i: (0, 0)),
        ],
        out_specs=[
            pl.BlockSpec((NB, H), lambda i: (0, 0)),
            pl.BlockSpec((NB, H), lambda i: (0, 0)),
        ],
        out_shape=[
            jax.ShapeDtypeStruct((NB, H), jnp.float32),
            jax.ShapeDtypeStruct((NB, H), jnp.float32),
        ],
    )(x, batch, w1, b1, w2row, b2)


def _poolB_call(x, batch, w1, b1, w2row, b2, gmax):
    npr = x.shape[0]
    nb = npr // RB

    def body(x_ref, bt_ref, w1_ref, b1_ref, w2_ref, b2_ref, gm_ref,
             se_ref, sx_ref):
        i = pl.program_id(0)
        xb = x_ref[...]
        g = _attg(xb, w1_ref[...], b1_ref[...], w2_ref[...], b2_ref[...])
        bt = bt_ref[...]
        ses, sxs = [], []
        for bb in range(NB):
            msk = bt == bb
            e = jnp.where(msk, jnp.exp(g - gm_ref[bb, 0]), 0.0)
            ses.append(jnp.sum(e, 0, keepdims=True))
            sxs.append(jnp.sum(e * xb, 0, keepdims=True))
        se = jnp.broadcast_to(jnp.concatenate(ses, 0), (NB, H))
        sx = jnp.concatenate(sxs, 0)

        @pl.when(i == 0)
        def _():
            se_ref[...] = se
            sx_ref[...] = sx

        @pl.when(i != 0)
        def _():
            se_ref[...] = se_ref[...] + se
            sx_ref[...] = sx_ref[...] + sx

    return pl.pallas_call(
        body,
        grid=(nb,),
        in_specs=[
            pl.BlockSpec((RB, H), lambda i: (i, 0)),
            pl.BlockSpec((RB, 1), lambda i: (i, 0)),
            pl.BlockSpec((H, H // 2), lambda i: (0, 0)),
            pl.BlockSpec((1, H // 2), lambda i: (0, 0)),
            pl.BlockSpec((1, H // 2), lambda i: (0, 0)),
            pl.BlockSpec((1, 1), lambda i: (0, 0)),
            pl.BlockSpec((NB, H), lambda i: (0, 0)),
        ],
        out_specs=[
            pl.BlockSpec((NB, H), lambda i: (0, 0)),
            pl.BlockSpec((NB, H), lambda i: (0, 0)),
        ],
        out_shape=[
            jax.ShapeDtypeStruct((NB, H), jnp.float32),
            jax.ShapeDtypeStruct((NB, H), jnp.float32),
        ],
    )(x, batch, w1, b1, w2row, b2, gmax)


def _head_call(vmax, cmax, vse, vsx, cse, csx, hp):
    def body(vm, cm, vse_r, vsx_r, cse_r, csx_r, w1, b1, w2, b2,
             wmu, bmu, wsg, bsg, o_ref):
        vatt = vsx_r[...] / (vse_r[...] + 1e-16)
        catt = csx_r[...] / (cse_r[...] + 1e-16)
        emb = jnp.concatenate([vm[...], cm[...], vatt, catt], axis=1)
        h = jnp.maximum(jnp.dot(emb, w1[...],
                                preferred_element_type=jnp.float32)
                        + b1[...], 0.0)
        h = jnp.maximum(jnp.dot(h, w2[...],
                                preferred_element_type=jnp.float32)
                        + b2[...], 0.0)
        mu = jnp.dot(h, wmu[...], preferred_element_type=jnp.float32) + bmu[...]
        sg = jnp.exp(jnp.dot(h, wsg[...], preferred_element_type=jnp.float32)
                     + bsg[...])
        o_ref[0] = mu
        o_ref[1] = sg

    out = pl.pallas_call(
        body,
        out_shape=jax.ShapeDtypeStruct((2, NB, CFG), jnp.float32),
    )(vmax, cmax, vse, vsx, cse, csx,
      hp['W1'], hp['b1'][None], hp['W2'], hp['b2'][None],
      hp['Wmu'], hp['bmu'][None], hp['Wsig'], hp['bsig'][None])
    return jnp.stack([out[0], out[1]], axis=-1)


def _pad_rows_cols(a, rows, cols):
    return jnp.pad(a, ((0, rows - a.shape[0]), (0, cols - a.shape[1])))


def kernel(var_feats, cstr_feats, edge_attr, edge_index, var_batch_el,
           cstr_batch_el, params):
    xv = _pad_rows_cols(var_feats, NVP, 16)
    xc = _pad_rows_cols(cstr_feats, NCP, 16)
    srcp = jnp.pad(edge_index[0], (0, EPAD - NEDGE),
                   constant_values=NCP - 1).reshape(EPAD // EB, EB)
    dstp = jnp.pad(edge_index[1], (0, EPAD - NEDGE),
                   constant_values=NCP - 1).reshape(EPAD // EB, EB)
    ewp = jnp.pad(edge_attr, (0, EPAD - NEDGE)).reshape(EPAD // EB, EB)

    deg_both = _deg_pass_call(dstp, srcp)
    degv2 = deg_both[:, 0]
    degc2 = deg_both[:, 1]

    for li, L in enumerate(params['layers']):
        d_v, d_c = xv.shape[1], xc.shape[1]
        if li == 0:
            ng = jnp.pad(L['nbn_g'], (0, 16 - 9), constant_values=1.0)[None]
            nb_ = jnp.pad(L['nbn_b'], (0, 16 - 9))[None]
            cg = jnp.pad(L['cbn_g'], (0, 16 - 1), constant_values=1.0)[None]
            cb_ = jnp.pad(L['cbn_b'], (0, 16 - 1))[None]
            nWroot = jnp.pad(L['nWroot'], ((0, 16 - 9), (0, 0)))
            cWroot = jnp.pad(L['cWroot'], ((0, 16 - 1), (0, 0)))
            nWrel = jnp.pad(L['nWrel'], ((0, 16 - 1), (0, 0)))
            cWrel = jnp.pad(L['cWrel'], ((0, 16 - 9), (0, 0)))
        else:
            ng, nb_ = L['nbn_g'][None], L['nbn_b'][None]
            cg, cb_ = L['cbn_g'][None], L['cbn_b'][None]
            nWroot, cWroot = L['nWroot'], L['cWroot']
            nWrel, cWrel = L['nWrel'], L['cWrel']

        stv = _stats_call(xv, d_v)
        stc = _stats_call(xc, d_c)
        xv_n, rootv = _norm_call(xv, stv, ng, nb_, nWroot, NVAR, d_v)
        xc_n, rootc = _norm_call(xc, stc, cg, cb_, cWroot, NCSTR, d_c)
        accv2 = _edge_pass_call(d_c, xc_n, srcp, dstp, ewp)
        accc2 = _edge_pass_call(d_v, xv_n, dstp, srcp, ewp)
        xv = _post_call(accv2, degv2, rootv, nWrel, L['nbrel'][None],
                        NVAR, d_c)
        xc = _post_call(accc2, degc2, rootc, cWrel, L['cbrel'][None],
                        NCSTR, d_v)

    vb = jnp.pad(var_batch_el, (0, NVP - NVAR), constant_values=NB)[:, None]
    cbt = jnp.pad(cstr_batch_el, (0, NCP - NCSTR), constant_values=NB)[:, None]
    av, ac = params['att_var'], params['att_cstr']
    aw2v = av['W2'][:, 0][None]
    aw2c = ac['W2'][:, 0][None]
    ab2v = av['b2'][None]
    ab2c = ac['b2'][None]
    vmax, vgm = _poolA_call(xv, vb, av['W1'], av['b1'][None], aw2v, ab2v)
    cmax, cgm = _poolA_call(xc, cbt, ac['W1'], ac['b1'][None], aw2c, ab2c)
    vse, vsx = _poolB_call(xv, vb, av['W1'], av['b1'][None], aw2v, ab2v, vgm)
    cse, csx = _poolB_call(xc, cbt, ac['W1'], ac['b1'][None], aw2c, ab2c, cgm)
    return _head_call(vmax, cmax, vse, vsx, cse, csx, params['head'])
